# baseline (speedup 1.0000x reference)
import jax
import jax.numpy as jnp
from jax import lax
from jax.experimental import pallas as pl
from jax.experimental.pallas import tpu as pltpu

_K = 5
_B = 1024
_N = 100000
_RB = 8


def _tc_body(prob_ref, lab_ref, out_ref, acc_ref):
    j = pl.program_id(0)
    p = prob_ref[...]
    lab = lab_ref[...]
    pi = lax.bitcast_convert_type(p, jnp.int32)
    col = lax.broadcasted_iota(jnp.int32, (_RB, _N), 1)
    q = pi + pi + (col < lab).astype(jnp.int32)
    vli = jnp.sum(jnp.where(col == lab, pi, 0), axis=1, keepdims=True)
    t1 = vli + vli
    t2 = pi[:, 0:1] * 2 + 1
    c1 = jnp.sum((q > t1).astype(jnp.int32), axis=1, keepdims=True)
    c2 = jnp.sum((q > t2).astype(jnp.int32), axis=1, keepdims=True)
    z = (c2 < _K).astype(jnp.float32)
    hit = ((c1 < _K) & (lab != 0)).astype(jnp.float32)
    f1 = 2.0 * hit / (_K + 1.0 - z)
    s = jnp.sum(f1)

    @pl.when(j == 0)
    def _():
        acc_ref[0] = s

    @pl.when(j > 0)
    def _():
        acc_ref[0] += s

    @pl.when(j == _B // _RB - 1)
    def _():
        out_ref[0, 0] = acc_ref[0] * (1.0 / _B)


def _tc_f1(prob, lab2d):
    return pl.pallas_call(
        _tc_body,
        grid=(_B // _RB,),
        in_specs=[
            pl.BlockSpec((_RB, _N), lambda j: (j, 0)),
            pl.BlockSpec((_RB, 1), lambda j: (j, 0)),
        ],
        out_specs=pl.BlockSpec(memory_space=pltpu.SMEM),
        out_shape=jax.ShapeDtypeStruct((1, 1), jnp.float32),
        scratch_shapes=[pltpu.SMEM((1,), jnp.float32)],
    )(prob, lab2d)


def kernel(prob, label):
    out = _tc_f1(prob, label.reshape(_B, 1))
    return out[0, 0]
